# baseline (speedup 1.0000x reference)
import functools

import jax
import jax.numpy as jnp
from jax import lax
from jax.experimental import pallas as pl
from jax.experimental.pallas import tpu as pltpu
from jax.experimental.pallas import tpu_sc as plsc

N = 10000
E = 320000
D = 128

NC = 2
NS = 16
NW = NC * NS

CHUNK = 128
TOT_CHUNKS = E // CHUNK
NCHF = 120
NCHS = 36
REM = TOT_CHUNKS - NS * (NCHF + NCHS)
FAST_TOT = NS * NCHF + REM
MAXCH = NCHF + 1

N_PAD = 10240
ROWS_PER_TILE = N_PAD // NS
GRID = 10
BLK = N_PAD // GRID

_mesh = plsc.VectorSubcoreMesh(core_axis_name="c", subcore_axis_name="s")


def _tile_span(c, s):
    base0 = s * NCHF + jnp.minimum(s, REM)
    nch0 = NCHF + jnp.where(s < REM, 1, 0)
    base1 = FAST_TOT + s * NCHS
    base = jnp.where(c == 0, base0, base1)
    nch = jnp.where(c == 0, nch0, NCHS)
    return base, nch


def _deg_body(edges3, degp, st_v, ones_v, zdeg_v, deg_sh, sem_s):
    c = lax.axis_index("c")
    s = lax.axis_index("s")
    base, nch = _tile_span(c, s)
    r0 = s * ROWS_PER_TILE

    z16 = jnp.zeros((16,), jnp.float32)
    for k in range(ROWS_PER_TILE // 16):
        zdeg_v[pl.ds(16 * k, 16)] = z16
    for k in range(CHUNK // 16):
        ones_v[pl.ds(16 * k, 16)] = jnp.full((16,), 1.0, jnp.float32)
    pltpu.sync_copy(zdeg_v, deg_sh.at[pl.ds(r0, ROWS_PER_TILE)])
    clb = jnp.minimum(base, TOT_CHUNKS - MAXCH)
    off = base - clb
    pltpu.sync_copy(edges3.at[pl.ds(clb, MAXCH)], st_v)
    plsc.subcore_barrier()

    WIN = 8

    def roll(j, carry):
        @pl.when(j < nch)
        def _():
            pltpu.async_copy(ones_v, deg_sh.at[st_v.at[off + j, 1]], sem_s,
                             add=True)

        @pl.when((j >= WIN) & (j - WIN < nch))
        def _():
            pltpu.make_async_copy(ones_v, deg_sh.at[st_v.at[0, 1]],
                                  sem_s).wait()

        return carry

    lax.fori_loop(0, MAXCH + WIN, roll, 0)
    plsc.subcore_barrier()
    pltpu.sync_copy(deg_sh.at[pl.ds(r0, ROWS_PER_TILE)],
                    degp.at[c, pl.ds(r0, ROWS_PER_TILE)])


_deg_kernel = functools.partial(
    pl.kernel,
    out_type=jax.ShapeDtypeStruct((NC, N_PAD), jnp.float32),
    mesh=_mesh,
    scratch_types=[
        pltpu.VMEM((MAXCH, 2, CHUNK), jnp.int32),
        pltpu.VMEM((CHUNK,), jnp.float32),
        pltpu.VMEM((ROWS_PER_TILE,), jnp.float32),
        pltpu.VMEM_SHARED((N_PAD,), jnp.float32),
        pltpu.SemaphoreType.DMA,
    ],
)(_deg_body)


def _scat_body(edges3, hp, accp, idxr, rows_v, acc_sh, sem_i, sem_g):
    c = lax.axis_index("c")
    s = lax.axis_index("s")
    base, nch = _tile_span(c, s)
    r0 = s * ROWS_PER_TILE

    z16 = jnp.zeros((16,), jnp.float32)
    for i in range(CHUNK):
        for k in range(D // 16):
            rows_v[0, i, pl.ds(16 * k, 16)] = z16
    for t in range(ROWS_PER_TILE // CHUNK):
        pltpu.sync_copy(rows_v.at[0],
                        acc_sh.at[pl.ds(r0 + t * CHUNK, CHUNK)])
    plsc.subcore_barrier()

    pltpu.sync_copy(edges3.at[base], idxr.at[0])
    pltpu.async_copy(edges3.at[base + 1], idxr.at[1], sem_i)
    pltpu.async_copy(hp.at[idxr.at[0, 0]], rows_v.at[0], sem_g)

    def body(j, carry):
        @pl.when(j < nch)
        def _():
            nxt = j + 1
            cur = lax.rem(j, 2)
            opp = lax.rem(nxt, 2)

            @pl.when(nxt < nch)
            def _():
                pltpu.make_async_copy(edges3.at[base + nxt], idxr.at[opp],
                                      sem_i).wait()
                pltpu.async_copy(hp.at[idxr.at[opp, 0]], rows_v.at[opp],
                                 sem_g)

            pltpu.make_async_copy(hp.at[idxr.at[cur, 0]], rows_v.at[cur],
                                  sem_g).wait()
            pltpu.sync_copy(rows_v.at[cur], acc_sh.at[idxr.at[cur, 1]],
                            add=True)

            @pl.when(j + 2 < nch)
            def _():
                pltpu.async_copy(edges3.at[base + j + 2], idxr.at[cur],
                                 sem_i)

        return carry

    lax.fori_loop(0, MAXCH, body, 0)
    plsc.subcore_barrier()
    pltpu.sync_copy(acc_sh.at[pl.ds(r0, ROWS_PER_TILE)],
                    accp.at[c, pl.ds(r0, ROWS_PER_TILE)])


_scat_kernel = functools.partial(
    pl.kernel,
    out_type=jax.ShapeDtypeStruct((NC, N_PAD, D), jnp.float32),
    mesh=_mesh,
    scratch_types=[
        pltpu.VMEM((2, 2, CHUNK), jnp.int32),
        pltpu.VMEM((2, CHUNK, D), jnp.float32),
        pltpu.VMEM_SHARED((N_PAD, D), jnp.float32),
        pltpu.SemaphoreType.DMA,
        pltpu.SemaphoreType.DMA,
    ],
)(_scat_body)


def _lin_body(x_ref, w_ref, degp_ref, hp_ref):
    deg = 1.0 + degp_ref[0, :] + degp_ref[1, :]
    dinv = lax.rsqrt(deg)[:, None]
    h = jnp.dot(x_ref[...], w_ref[...], preferred_element_type=jnp.float32)
    hp_ref[...] = h * dinv


def _comb_body(accp_ref, hp_ref, degp_ref, b_ref, out_ref):
    deg = 1.0 + degp_ref[0, :] + degp_ref[1, :]
    dinv = lax.rsqrt(deg)[:, None]
    out_ref[...] = dinv * (accp_ref[0] + accp_ref[1] + hp_ref[...]) + b_ref[...]


def kernel(x, edge_index, W, b):
    ei = edge_index.astype(jnp.int32)
    edges3 = jnp.stack([ei[0].reshape(TOT_CHUNKS, CHUNK),
                        ei[1].reshape(TOT_CHUNKS, CHUNK)], axis=1)

    degp = _deg_kernel(edges3)

    hp = pl.pallas_call(
        _lin_body,
        grid=(GRID,),
        in_specs=[
            pl.BlockSpec((BLK, D), lambda i: (i, 0)),
            pl.BlockSpec((D, D), lambda i: (0, 0)),
            pl.BlockSpec((NC, BLK), lambda i: (0, i)),
        ],
        out_specs=pl.BlockSpec((BLK, D), lambda i: (i, 0)),
        out_shape=jax.ShapeDtypeStruct((N, D), jnp.float32),
    )(x, W, degp)

    accp = _scat_kernel(edges3, hp)

    out = pl.pallas_call(
        _comb_body,
        grid=(GRID,),
        in_specs=[
            pl.BlockSpec((NC, BLK, D), lambda i: (0, i, 0)),
            pl.BlockSpec((BLK, D), lambda i: (i, 0)),
            pl.BlockSpec((NC, BLK), lambda i: (0, i)),
            pl.BlockSpec((1, D), lambda i: (0, 0)),
        ],
        out_specs=pl.BlockSpec((BLK, D), lambda i: (i, 0)),
        out_shape=jax.ShapeDtypeStruct((N, D), jnp.float32),
    )(accp, hp, degp, b.reshape(1, D))

    return out


# speedup vs baseline: 1.1289x; 1.1289x over previous
import functools

import jax
import jax.numpy as jnp
from jax import lax
from jax.experimental import pallas as pl
from jax.experimental.pallas import tpu as pltpu
from jax.experimental.pallas import tpu_sc as plsc

N = 10000
E = 320000
D = 128

NC = 2
NS = 16
NW = NC * NS

CHUNK = 128
TOT_CHUNKS = E // CHUNK
NCHF = 103
NCHS = 53
REM = TOT_CHUNKS - NS * (NCHF + NCHS)
FAST_TOT = NS * NCHF + REM
MAXCH = NCHF + 1

N_PAD = 10240
ROWS_PER_TILE = N_PAD // NS
GRID = 10
BLK = N_PAD // GRID

_mesh = plsc.VectorSubcoreMesh(core_axis_name="c", subcore_axis_name="s")


def _tile_span(c, s):
    base0 = s * NCHF + jnp.minimum(s, REM)
    nch0 = NCHF + jnp.where(s < REM, 1, 0)
    base1 = FAST_TOT + s * NCHS
    base = jnp.where(c == 0, base0, base1)
    nch = jnp.where(c == 0, nch0, NCHS)
    return base, nch


def _deg_body(ei, degp, st_v, ones_v, zdeg_v, deg_sh, sem_i, sem_s):
    c = lax.axis_index("c")
    s = lax.axis_index("s")
    base, nch = _tile_span(c, s)
    r0 = s * ROWS_PER_TILE

    z16 = jnp.zeros((16,), jnp.float32)
    for k in range(ROWS_PER_TILE // 16):
        zdeg_v[pl.ds(16 * k, 16)] = z16
    for k in range(CHUNK // 16):
        ones_v[pl.ds(16 * k, 16)] = jnp.full((16,), 1.0, jnp.float32)
    pltpu.sync_copy(zdeg_v, deg_sh.at[pl.ds(r0, ROWS_PER_TILE)])
    clb = jnp.minimum(base, TOT_CHUNKS - MAXCH)
    off = base - clb
    WINL = 16

    def stage(r, carry):
        @pl.when(r < MAXCH)
        def _():
            o = pl.multiple_of((clb + r) * CHUNK, CHUNK)
            pltpu.async_copy(ei.at[1, pl.ds(o, CHUNK)], st_v.at[r], sem_i)

        @pl.when(r >= WINL)
        def _():
            pltpu.make_async_copy(ei.at[1, pl.ds(0, CHUNK)], st_v.at[0],
                                  sem_i).wait()

        return carry

    lax.fori_loop(0, MAXCH + WINL, stage, 0)
    plsc.subcore_barrier()

    WIN = 8

    def roll(j, carry):
        @pl.when(j < nch)
        def _():
            pltpu.async_copy(ones_v, deg_sh.at[st_v.at[off + j]], sem_s,
                             add=True)

        @pl.when((j >= WIN) & (j - WIN < nch))
        def _():
            pltpu.make_async_copy(ones_v, deg_sh.at[st_v.at[0]],
                                  sem_s).wait()

        return carry

    lax.fori_loop(0, MAXCH + WIN, roll, 0)
    plsc.subcore_barrier()
    pltpu.sync_copy(deg_sh.at[pl.ds(r0, ROWS_PER_TILE)],
                    degp.at[c, pl.ds(r0, ROWS_PER_TILE)])


_deg_kernel = functools.partial(
    pl.kernel,
    out_type=jax.ShapeDtypeStruct((NC, N_PAD), jnp.float32),
    mesh=_mesh,
    scratch_types=[
        pltpu.VMEM((MAXCH, CHUNK), jnp.int32),
        pltpu.VMEM((CHUNK,), jnp.float32),
        pltpu.VMEM((ROWS_PER_TILE,), jnp.float32),
        pltpu.VMEM_SHARED((N_PAD,), jnp.float32),
        pltpu.SemaphoreType.DMA,
        pltpu.SemaphoreType.DMA,
    ],
)(_deg_body)


def _scat_body(ei, hp, accp, idxr, rows_v, acc_sh, sem_i, sem_g):
    c = lax.axis_index("c")
    s = lax.axis_index("s")
    base, nch = _tile_span(c, s)
    r0 = s * ROWS_PER_TILE

    z16 = jnp.zeros((16,), jnp.float32)
    for i in range(CHUNK):
        for k in range(D // 16):
            rows_v[0, i, pl.ds(16 * k, 16)] = z16
    for t in range(ROWS_PER_TILE // CHUNK):
        pltpu.sync_copy(rows_v.at[0],
                        acc_sh.at[pl.ds(r0 + t * CHUNK, CHUNK)])
    plsc.subcore_barrier()

    def load_idx(g, k):
        o = pl.multiple_of(g * CHUNK, CHUNK)
        pltpu.async_copy(ei.at[0, pl.ds(o, CHUNK)], idxr.at[k, 0], sem_i)
        pltpu.async_copy(ei.at[1, pl.ds(o, CHUNK)], idxr.at[k, 1], sem_i)

    def wait_idx(k):
        pltpu.make_async_copy(ei.at[0, pl.ds(0, CHUNK)], idxr.at[k, 0],
                              sem_i).wait()
        pltpu.make_async_copy(ei.at[1, pl.ds(0, CHUNK)], idxr.at[k, 1],
                              sem_i).wait()

    load_idx(base, 0)
    wait_idx(0)
    load_idx(base + 1, 1)
    pltpu.async_copy(hp.at[idxr.at[0, 0]], rows_v.at[0], sem_g)

    def body(j, carry):
        @pl.when(j < nch)
        def _():
            nxt = j + 1
            cur = lax.rem(j, 2)
            opp = lax.rem(nxt, 2)

            @pl.when(nxt < nch)
            def _():
                wait_idx(opp)
                pltpu.async_copy(hp.at[idxr.at[opp, 0]], rows_v.at[opp],
                                 sem_g)

            pltpu.make_async_copy(hp.at[idxr.at[cur, 0]], rows_v.at[cur],
                                  sem_g).wait()
            pltpu.sync_copy(rows_v.at[cur], acc_sh.at[idxr.at[cur, 1]],
                            add=True)

            @pl.when(j + 2 < nch)
            def _():
                load_idx(base + j + 2, cur)

        return carry

    lax.fori_loop(0, MAXCH, body, 0)
    plsc.subcore_barrier()
    pltpu.sync_copy(acc_sh.at[pl.ds(r0, ROWS_PER_TILE)],
                    accp.at[c, pl.ds(r0, ROWS_PER_TILE)])


_scat_kernel = functools.partial(
    pl.kernel,
    out_type=jax.ShapeDtypeStruct((NC, N_PAD, D), jnp.float32),
    mesh=_mesh,
    scratch_types=[
        pltpu.VMEM((2, 2, CHUNK), jnp.int32),
        pltpu.VMEM((2, CHUNK, D), jnp.float32),
        pltpu.VMEM_SHARED((N_PAD, D), jnp.float32),
        pltpu.SemaphoreType.DMA,
        pltpu.SemaphoreType.DMA,
    ],
)(_scat_body)


def _lin_body(x_ref, w_ref, degp_ref, hp_ref):
    deg = 1.0 + degp_ref[0, :] + degp_ref[1, :]
    dinv = lax.rsqrt(deg)[:, None]
    h = jnp.dot(x_ref[...], w_ref[...], preferred_element_type=jnp.float32)
    hp_ref[...] = h * dinv


def _comb_body(accp_ref, hp_ref, degp_ref, b_ref, out_ref):
    deg = 1.0 + degp_ref[0, :] + degp_ref[1, :]
    dinv = lax.rsqrt(deg)[:, None]
    out_ref[...] = dinv * (accp_ref[0] + accp_ref[1] + hp_ref[...]) + b_ref[...]


def kernel(x, edge_index, W, b):
    ei = edge_index.astype(jnp.int32)

    degp = _deg_kernel(ei)

    hp = pl.pallas_call(
        _lin_body,
        grid=(GRID,),
        in_specs=[
            pl.BlockSpec((BLK, D), lambda i: (i, 0)),
            pl.BlockSpec((D, D), lambda i: (0, 0)),
            pl.BlockSpec((NC, BLK), lambda i: (0, i)),
        ],
        out_specs=pl.BlockSpec((BLK, D), lambda i: (i, 0)),
        out_shape=jax.ShapeDtypeStruct((N, D), jnp.float32),
    )(x, W, degp)

    accp = _scat_kernel(ei, hp)

    out = pl.pallas_call(
        _comb_body,
        grid=(GRID,),
        in_specs=[
            pl.BlockSpec((NC, BLK, D), lambda i: (0, i, 0)),
            pl.BlockSpec((BLK, D), lambda i: (i, 0)),
            pl.BlockSpec((NC, BLK), lambda i: (0, i)),
            pl.BlockSpec((1, D), lambda i: (0, 0)),
        ],
        out_specs=pl.BlockSpec((BLK, D), lambda i: (i, 0)),
        out_shape=jax.ShapeDtypeStruct((N, D), jnp.float32),
    )(accp, hp, degp, b.reshape(1, D))

    return out
